# baseline (speedup 1.0000x reference)
import functools

import jax
import jax.numpy as jnp
from jax import lax
from jax.experimental import pallas as pl
from jax.experimental.pallas import tpu as pltpu
from jax.experimental.pallas import tpu_sc as plsc

N_SAE = 8
D_DATA = 256
D_DICT = 1024
TOKENS = 2048
TB = 256


def _dense_body(x_ref, g_ref, we_ref, be_ref, wd_ref, bd_ref, o_ref):
    x = x_ref[...].astype(jnp.bfloat16)
    g = g_ref[...]
    acc = jnp.zeros((TB, D_DATA), jnp.float32)
    for s in range(N_SAE):
        acts = jnp.maximum(
            jnp.dot(x, we_ref[s], preferred_element_type=jnp.float32)
            + be_ref[s][None, :],
            0.0,
        )
        gs = g[:, s:s + 1]
        dec = jnp.dot((acts * gs).astype(jnp.bfloat16),
                      wd_ref[s], preferred_element_type=jnp.float32)
        msk = (gs != 0.0).astype(jnp.float32)
        acc = acc + dec + msk * bd_ref[s][None, :]
    o_ref[...] = acc


def kernel(x, gate, W_enc, b_enc, W_dec, b_dec):
    grid = (TOKENS // TB,)
    out = pl.pallas_call(
        _dense_body,
        grid=grid,
        in_specs=[
            pl.BlockSpec((TB, D_DATA), lambda i: (i, 0)),
            pl.BlockSpec((TB, N_SAE), lambda i: (i, 0)),
            pl.BlockSpec((N_SAE, D_DATA, D_DICT), lambda i: (0, 0, 0)),
            pl.BlockSpec((N_SAE, D_DICT), lambda i: (0, 0)),
            pl.BlockSpec((N_SAE, D_DICT, D_DATA), lambda i: (0, 0, 0)),
            pl.BlockSpec((N_SAE, D_DATA), lambda i: (0, 0)),
        ],
        out_specs=pl.BlockSpec((TB, D_DATA), lambda i: (i, 0)),
        out_shape=jax.ShapeDtypeStruct((TOKENS, D_DATA), jnp.float32),
        compiler_params=pltpu.CompilerParams(
            dimension_semantics=("parallel",),
        ),
    )(x, gate, W_enc.astype(jnp.bfloat16), b_enc,
      W_dec.astype(jnp.bfloat16), b_dec)
    return out


# speedup vs baseline: 1.2634x; 1.2634x over previous
import functools

import jax
import jax.numpy as jnp
from jax import lax
from jax.experimental import pallas as pl
from jax.experimental.pallas import tpu as pltpu
from jax.experimental.pallas import tpu_sc as plsc

N_SAE = 8
D_DATA = 256
D_DICT = 1024
TOKENS = 2048
TB = 256


def _dense_body(x_ref, g_ref, we_ref, be_ref, wd_ref, bd_ref, o_ref,
                webf, wdbf):
    @pl.when(pl.program_id(0) == 0)
    def _cast_once():
        webf[...] = we_ref[...].astype(jnp.bfloat16)
        wdbf[...] = wd_ref[...].astype(jnp.bfloat16)

    x = x_ref[...].astype(jnp.bfloat16)
    g = g_ref[...]
    acc = jnp.zeros((TB, D_DATA), jnp.float32)
    for s in range(N_SAE):
        acts = jnp.maximum(
            jnp.dot(x, webf[s], preferred_element_type=jnp.float32)
            + be_ref[s][None, :],
            0.0,
        )
        gs = g[:, s:s + 1]
        dec = jnp.dot((acts * gs).astype(jnp.bfloat16),
                      wdbf[s], preferred_element_type=jnp.float32)
        msk = (gs != 0.0).astype(jnp.float32)
        acc = acc + dec + msk * bd_ref[s][None, :]
    o_ref[...] = acc


def kernel(x, gate, W_enc, b_enc, W_dec, b_dec):
    grid = (TOKENS // TB,)
    out = pl.pallas_call(
        _dense_body,
        grid=grid,
        in_specs=[
            pl.BlockSpec((TB, D_DATA), lambda i: (i, 0)),
            pl.BlockSpec((TB, N_SAE), lambda i: (i, 0)),
            pl.BlockSpec((N_SAE, D_DATA, D_DICT), lambda i: (0, 0, 0)),
            pl.BlockSpec((N_SAE, D_DICT), lambda i: (0, 0)),
            pl.BlockSpec((N_SAE, D_DICT, D_DATA), lambda i: (0, 0, 0)),
            pl.BlockSpec((N_SAE, D_DATA), lambda i: (0, 0)),
        ],
        out_specs=pl.BlockSpec((TB, D_DATA), lambda i: (i, 0)),
        out_shape=jax.ShapeDtypeStruct((TOKENS, D_DATA), jnp.float32),
        scratch_shapes=[
            pltpu.VMEM((N_SAE, D_DATA, D_DICT), jnp.bfloat16),
            pltpu.VMEM((N_SAE, D_DICT, D_DATA), jnp.bfloat16),
        ],
        compiler_params=pltpu.CompilerParams(
            dimension_semantics=("arbitrary",),
        ),
    )(x, gate, W_enc, b_enc, W_dec, b_dec)
    return out
